# Initial kernel scaffold; baseline (speedup 1.0000x reference)
import jax
import jax.numpy as jnp
from jax.experimental import pallas as pl


def kernel(x, router_w, router_b, W1, W2, W3):
    raise NotImplementedError("write your pallas kernel here")


# speedup vs baseline: 1.3777x; 1.3777x over previous
import functools

import jax
import jax.numpy as jnp
from jax.experimental import pallas as pl
from jax.experimental.pallas import tpu as pltpu

T = 2048
D = 1024
E = 8
H = 2816
HC = 2
Hc = H // HC


def _router_body(x_ref, rw_ref, rb_ref, comb_ref, aux_ref):
    x = x_ref[...]
    rw = rw_ref[...]
    logits = jax.lax.dot_general(x, rw, (((1,), (1,)), ((), ())),
                                 preferred_element_type=jnp.float32)
    logits = logits + rb_ref[...][None, :]
    nrm = jnp.sqrt(jnp.sum(logits * logits, axis=0, keepdims=True))
    rl = logits / jnp.maximum(nrm, 1e-12)
    m = jnp.max(rl, axis=-1, keepdims=True)
    ex = jnp.exp(rl - m)
    probs = ex / jnp.sum(ex, axis=-1, keepdims=True)
    lane = jax.lax.broadcasted_iota(jnp.int32, (T, E), 1)
    m1 = jnp.max(probs, axis=-1, keepdims=True)
    i1 = jnp.min(jnp.where(probs == m1, lane, E), axis=-1, keepdims=True)
    mask1 = lane == i1
    p2 = jnp.where(mask1, -jnp.inf, probs)
    m2 = jnp.max(p2, axis=-1, keepdims=True)
    i2 = jnp.min(jnp.where(p2 == m2, lane, E), axis=-1, keepdims=True)
    comb = jnp.where(mask1, m1, 0.0) + jnp.where(lane == i2, m2, 0.0)
    comb_ref[...] = comb
    aux = jnp.sum((1.0 / E - probs) ** 2)
    aux_ref[0, 0] = aux


def _router(x2d, router_w, router_b):
    return pl.pallas_call(
        _router_body,
        out_shape=(
            jax.ShapeDtypeStruct((T, E), jnp.float32),
            jax.ShapeDtypeStruct((1, 1), jnp.float32),
        ),
        in_specs=[
            pl.BlockSpec((T, D), lambda: (0, 0)),
            pl.BlockSpec((E, D), lambda: (0, 0)),
            pl.BlockSpec((E,), lambda: (0,)),
        ],
        out_specs=(
            pl.BlockSpec((T, E), lambda: (0, 0)),
            pl.BlockSpec((1, 1), memory_space=pltpu.SMEM),
        ),
    )(x2d, router_w, router_b)


TB = 256
NT = T // TB


def _ffn_body(x_ref, w1_ref, w3_ref, w2_ref, comb_ref, out_ref):
    e = pl.program_id(0)
    h = pl.program_id(1)
    t = pl.program_id(2)

    @pl.when(jnp.logical_and(jnp.logical_and(e == 0, h == 0), t == 0))
    def _():
        out_ref[...] = jnp.zeros_like(out_ref)

    x = x_ref[...]
    h1 = jax.lax.dot_general(x, w1_ref[0], (((1,), (1,)), ((), ())),
                             preferred_element_type=jnp.float32)
    h3 = jax.lax.dot_general(x, w3_ref[0], (((1,), (1,)), ((), ())),
                             preferred_element_type=jnp.float32)
    g = jnp.sin(h1) * h3
    oe = jax.lax.dot_general(g, w2_ref[0], (((1,), (1,)), ((), ())),
                             preferred_element_type=jnp.float32)
    lane = jax.lax.broadcasted_iota(jnp.int32, (TB, E), 1)
    scale = jnp.sum(jnp.where(lane == e, comb_ref[...], 0.0), axis=1,
                    keepdims=True)
    out_ref[pl.ds(t * TB, TB), :] += oe * scale


def _ffn(x2d, W1, W2, W3, comb):
    return pl.pallas_call(
        _ffn_body,
        grid=(E, HC, NT),
        out_shape=jax.ShapeDtypeStruct((T, D), jnp.float32),
        in_specs=[
            pl.BlockSpec((TB, D), lambda e, h, t: (t, 0)),
            pl.BlockSpec((1, Hc, D), lambda e, h, t: (e, h, 0)),
            pl.BlockSpec((1, Hc, D), lambda e, h, t: (e, h, 0)),
            pl.BlockSpec((1, D, Hc), lambda e, h, t: (e, 0, h)),
            pl.BlockSpec((TB, E), lambda e, h, t: (t, 0)),
        ],
        out_specs=pl.BlockSpec((T, D), lambda e, h, t: (0, 0)),
    )(x2d, W1, W3, W2, comb)


def kernel(x, router_w, router_b, W1, W2, W3):
    Bb, Ss, Dd = x.shape
    x2d = x.reshape(T, D)
    comb, aux = _router(x2d, router_w, router_b)
    out = _ffn(x2d, W1, W2, W3, comb)
    return out.reshape(Bb, Ss, Dd), aux.reshape(())


# speedup vs baseline: 2.4149x; 1.7529x over previous
import functools

import jax
import jax.numpy as jnp
from jax import lax
from jax.experimental import pallas as pl
from jax.experimental.pallas import tpu as pltpu
from jax.experimental.pallas import tpu_sc as plsc

T = 2048
D = 1024
E = 8
H = 2816
K = 2
A = T * K
BLK = 512
LOG2_BLK = 9
NB = 16
NTOT = NB * BLK
HC = 2
Hc = H // HC

_SC_MESH = plsc.VectorSubcoreMesh(core_axis_name="c", subcore_axis_name="s")
_NW = 32
_SC_PARAMS = pltpu.CompilerParams(needs_layout_passes=False)


def _router_body(x_ref, rw_ref, rb_ref, ids_ref, w_ref, aux_ref):
    x = x_ref[...]
    rw = rw_ref[...]
    logits = lax.dot_general(x, rw, (((1,), (1,)), ((), ())),
                             preferred_element_type=jnp.float32)
    logits = logits + rb_ref[...][None, :]
    nrm = jnp.sqrt(jnp.sum(logits * logits, axis=0, keepdims=True))
    rl = logits / jnp.maximum(nrm, 1e-12)
    m = jnp.max(rl, axis=-1, keepdims=True)
    ex = jnp.exp(rl - m)
    probs = ex / jnp.sum(ex, axis=-1, keepdims=True)
    lane = lax.broadcasted_iota(jnp.int32, (T, E), 1)
    m1 = jnp.max(probs, axis=-1, keepdims=True)
    i1 = jnp.min(jnp.where(probs == m1, lane, E), axis=-1, keepdims=True)
    mask1 = lane == i1
    p2 = jnp.where(mask1, -jnp.inf, probs)
    m2 = jnp.max(p2, axis=-1, keepdims=True)
    i2 = jnp.min(jnp.where(p2 == m2, lane, E), axis=-1, keepdims=True)
    ids_ref[...] = jnp.concatenate([i1, i2], axis=1)
    w_ref[...] = jnp.concatenate([m1, m2], axis=1)
    aux_ref[0, 0] = jnp.sum((1.0 / E - probs) ** 2)


def _router(x2d, router_w, router_b):
    return pl.pallas_call(
        _router_body,
        out_shape=(
            jax.ShapeDtypeStruct((T, K), jnp.int32),
            jax.ShapeDtypeStruct((T, K), jnp.float32),
            jax.ShapeDtypeStruct((1, 1), jnp.float32),
        ),
        in_specs=[
            pl.BlockSpec((T, D), lambda: (0, 0)),
            pl.BlockSpec((E, D), lambda: (0, 0)),
            pl.BlockSpec((E,), lambda: (0,)),
        ],
        out_specs=(
            pl.BlockSpec((T, K), lambda: (0, 0)),
            pl.BlockSpec((T, K), lambda: (0, 0)),
            pl.BlockSpec((1, 1), memory_space=pltpu.SMEM),
        ),
    )(x2d, router_w, router_b)


def _dispatch_body(ids_hbm, w_hbm, st_hbm, ws_hbm, pos_hbm, be_hbm, nr_hbm,
                  ids_v, w_v, st_v, ws_v, pos_v, cnt_v, off_v, sblk_v, tmp_v):
    wid = lax.axis_index("s") * 2 + lax.axis_index("c")

    @pl.when(wid == 0)
    def _():
        pltpu.sync_copy(ids_hbm, ids_v)
        pltpu.sync_copy(w_hbm, w_v)
        ones = jnp.ones((16,), jnp.int32)
        zeros = jnp.zeros((16,), jnp.int32)
        lane16 = lax.iota(jnp.int32, 16)

        cnt_v[...] = zeros

        def count_step(i, _):
            ids16 = ids_v[pl.ds(i * 16, 16)]
            plsc.addupdate_scatter(cnt_v, [ids16], ones)
            return 0

        lax.fori_loop(0, A // 16, count_step, 0)

        cnt = cnt_v[...]
        nblk = (cnt + (BLK - 1)) >> LOG2_BLK
        pad = nblk << LOG2_BLK
        off = plsc.cumsum(pad) - pad
        off_v[...] = off
        sblk_v[...] = off >> LOG2_BLK

        acc = zeros
        for e in range(E):
            s_e = plsc.load_gather(sblk_v, [jnp.full((16,), e, jnp.int32)])
            acc = acc + jnp.where(lane16 >= s_e, 1, 0)
        be = jnp.minimum(acc - 1, E - 1)
        tmp_v[...] = be
        cnt_be = plsc.load_gather(cnt_v, [be])
        sblk_be = plsc.load_gather(sblk_v, [be])
        nrows = jnp.clip(cnt_be - ((lane16 - sblk_be) << LOG2_BLK), 0, BLK)
        pltpu.sync_copy(tmp_v, be_hbm)
        tmp_v[...] = nrows
        pltpu.sync_copy(tmp_v, nr_hbm)

        def zero_step(i, _):
            st_v[pl.ds(i * 16, 16)] = zeros
            ws_v[pl.ds(i * 16, 16)] = jnp.zeros((16,), jnp.float32)
            return 0

        lax.fori_loop(0, NTOT // 16, zero_step, 0)

        cnt_v[...] = zeros

        def place_step(i, _):
            a0 = i * 16
            ids16 = ids_v[pl.ds(a0, 16)]
            w16 = w_v[pl.ds(a0, 16)]
            rank1, _last = plsc.scan_count(ids16)
            base = plsc.load_gather(cnt_v, [ids16])
            offg = plsc.load_gather(off_v, [ids16])
            pos16 = offg + base + rank1 - 1
            tok16 = (a0 + lane16) & (T - 1)
            plsc.store_scatter(st_v, [pos16], tok16)
            plsc.store_scatter(ws_v, [pos16], w16)
            pos_v[pl.ds(a0, 16)] = pos16
            plsc.addupdate_scatter(cnt_v, [ids16], ones)
            return 0

        lax.fori_loop(0, A // 16, place_step, 0)

        pltpu.sync_copy(st_v, st_hbm)
        pltpu.sync_copy(ws_v, ws_hbm)
        pltpu.sync_copy(pos_v, pos_hbm)


def _dispatch(ids_cm, w_cm):
    f = pl.kernel(
        _dispatch_body,
        out_type=(
            jax.ShapeDtypeStruct((NTOT,), jnp.int32),
            jax.ShapeDtypeStruct((NTOT,), jnp.float32),
            jax.ShapeDtypeStruct((A,), jnp.int32),
            jax.ShapeDtypeStruct((16,), jnp.int32),
            jax.ShapeDtypeStruct((16,), jnp.int32),
        ),
        mesh=_SC_MESH,
        scratch_types=[
            pltpu.VMEM((A,), jnp.int32),
            pltpu.VMEM((A,), jnp.float32),
            pltpu.VMEM((NTOT,), jnp.int32),
            pltpu.VMEM((NTOT,), jnp.float32),
            pltpu.VMEM((A,), jnp.int32),
            pltpu.VMEM((16,), jnp.int32),
            pltpu.VMEM((16,), jnp.int32),
            pltpu.VMEM((16,), jnp.int32),
            pltpu.VMEM((16,), jnp.int32),
        ],
        compiler_params=_SC_PARAMS,
    )
    return f(ids_cm, w_cm)


_GROWS = NTOT // _NW
_GCH = 64


def _gather_body(x_hbm, st_hbm, xs_hbm, idx_v, rows_v, sem):
    wid = lax.axis_index("s") * 2 + lax.axis_index("c")
    base = wid * _GROWS

    def step(c, _):
        r0 = base + c * _GCH
        pltpu.sync_copy(st_hbm.at[pl.ds(r0, _GCH)], idx_v)
        pltpu.async_copy(x_hbm.at[idx_v], rows_v, sem).wait()
        pltpu.sync_copy(rows_v, xs_hbm.at[pl.ds(r0, _GCH)])
        return 0

    lax.fori_loop(0, _GROWS // _GCH, step, 0)


def _gather(x2d, sorted_tok):
    f = pl.kernel(
        _gather_body,
        out_type=jax.ShapeDtypeStruct((NTOT, D), jnp.float32),
        mesh=_SC_MESH,
        scratch_types=[
            pltpu.VMEM((_GCH,), jnp.int32),
            pltpu.VMEM((_GCH, D), jnp.float32),
            pltpu.SemaphoreType.DMA,
        ],
        compiler_params=_SC_PARAMS,
    )
    return f(x2d, sorted_tok)


def _ffn_body(be_ref, nr_ref, xs_ref, w1_ref, w3_ref, w2_ref, ws_ref, out_ref):
    b = pl.program_id(0)
    h = pl.program_id(1)

    @pl.when(h == 0)
    def _():
        out_ref[...] = jnp.zeros_like(out_ref)

    @pl.when(nr_ref[b] > 0)
    def _():
        x = xs_ref[...]
        h1 = lax.dot_general(x, w1_ref[0], (((1,), (1,)), ((), ())),
                             preferred_element_type=jnp.float32)
        h3 = lax.dot_general(x, w3_ref[0], (((1,), (1,)), ((), ())),
                             preferred_element_type=jnp.float32)
        g = jnp.sin(h1) * h3 * ws_ref[0]
        out_ref[...] += lax.dot_general(g, w2_ref[0], (((1,), (1,)), ((), ())),
                                        preferred_element_type=jnp.float32)


def _ffn(xs, W1, W2, W3, ws_blk, be, nrows):
    grid_spec = pltpu.PrefetchScalarGridSpec(
        num_scalar_prefetch=2,
        grid=(NB, HC),
        in_specs=[
            pl.BlockSpec((BLK, D), lambda b, h, be, nr: (b, 0)),
            pl.BlockSpec((1, Hc, D), lambda b, h, be, nr: (be[b], h, 0)),
            pl.BlockSpec((1, Hc, D), lambda b, h, be, nr: (be[b], h, 0)),
            pl.BlockSpec((1, D, Hc), lambda b, h, be, nr: (be[b], 0, h)),
            pl.BlockSpec((1, BLK, 1), lambda b, h, be, nr: (b, 0, 0)),
        ],
        out_specs=pl.BlockSpec((BLK, D), lambda b, h, be, nr: (b, 0)),
    )
    return pl.pallas_call(
        _ffn_body,
        grid_spec=grid_spec,
        out_shape=jax.ShapeDtypeStruct((NTOT, D), jnp.float32),
    )(be, nrows, xs, W1, W3, W2, ws_blk)


_CTOK = T // _NW
_CCH = 32


def _combine_body(y_hbm, pos_hbm, out_hbm, idx_v, buf0_v, buf1_v, sem):
    wid = lax.axis_index("s") * 2 + lax.axis_index("c")
    base = wid * _CTOK

    def step(c, _):
        t0 = base + c * _CCH
        pltpu.sync_copy(pos_hbm.at[pl.ds(t0, _CCH)], idx_v)
        pltpu.async_copy(y_hbm.at[idx_v], buf0_v, sem).wait()
        pltpu.sync_copy(pos_hbm.at[pl.ds(T + t0, _CCH)], idx_v)
        pltpu.async_copy(y_hbm.at[idx_v], buf1_v, sem).wait()

        def add_step(i, _):
            r = i >> 6
            c16 = (i & 63) * 16
            buf0_v[r, pl.ds(c16, 16)] += buf1_v[r, pl.ds(c16, 16)]
            return 0

        lax.fori_loop(0, _CCH * (D // 16), add_step, 0)
        pltpu.sync_copy(buf0_v, out_hbm.at[pl.ds(t0, _CCH)])
        return 0

    lax.fori_loop(0, _CTOK // _CCH, step, 0)


def _combine(y, pos):
    f = pl.kernel(
        _combine_body,
        out_type=jax.ShapeDtypeStruct((T, D), jnp.float32),
        mesh=_SC_MESH,
        scratch_types=[
            pltpu.VMEM((_CCH,), jnp.int32),
            pltpu.VMEM((_CCH, D), jnp.float32),
            pltpu.VMEM((_CCH, D), jnp.float32),
            pltpu.SemaphoreType.DMA,
        ],
        compiler_params=_SC_PARAMS,
    )
    return f(y, pos)


def kernel(x, router_w, router_b, W1, W2, W3):
    Bb, Ss, Dd = x.shape
    x2d = x.reshape(T, D)
    ids, w, aux = _router(x2d, router_w, router_b)
    ids_cm = ids.T.reshape(A)
    w_cm = w.T.reshape(A)
    sorted_tok, w_sorted, pos, be, nrows = _dispatch(ids_cm, w_cm)
    xs = _gather(x2d, sorted_tok)
    ws_blk = w_sorted.reshape(NB, BLK, 1)
    y = _ffn(xs, W1, W2, W3, ws_blk, be, nrows)
    out = _combine(y, pos)
    return out.reshape(Bb, Ss, Dd), aux.reshape(())
